# baseline (speedup 1.0000x reference)
import jax
import jax.numpy as jnp
from jax.experimental import pallas as pl
from jax.experimental.pallas import tpu as pltpu

_NSPLIT = 2
_TB = 16384


def _fused_policy_kernel(*refs):
    x_refs = refs[:_NSPLIT]
    w1_ref, b1_ref, w2t_ref, b2_ref, wh_ref, bh_ref, out_ref = refs[_NSPLIT:]
    tbs = x_refs[0].shape[0]

    for j, x_ref in enumerate(x_refs):
        xb = x_ref[...].astype(jnp.bfloat16)

        h = jax.lax.dot_general(
            w1_ref[...], xb, (((0,), (1,)), ((), ())),
            preferred_element_type=jnp.float32) + b1_ref[...]
        h = jnp.maximum(h, 0.0).astype(jnp.bfloat16)

        h = jnp.dot(w2t_ref[...], h,
                    preferred_element_type=jnp.float32) + b2_ref[...]
        h = jnp.maximum(h, 0.0).astype(jnp.bfloat16)

        z = jnp.dot(wh_ref[...], h,
                    preferred_element_type=jnp.float32) + bh_ref[...]

        mu_all = jnp.tanh(z) * 2.0
        sig_all = (jnp.maximum(z, 0.0)
                   + jnp.log1p(jnp.exp(-jnp.abs(z)))
                   + 1e-5)
        row = jax.lax.broadcasted_iota(jnp.int32, z.shape, dimension=0)
        out_ref[:, pl.ds(j * tbs, tbs)] = jnp.where(row == 0, mu_all, sig_all)


def kernel(x, w1, b1, w2, b2, w_mu, b_mu, w_sig, b_sig):
    B, S = x.shape
    H = w1.shape[1]

    w1b = w1.astype(jnp.bfloat16)
    b1t = b1.reshape(H, 1)
    w2tb = w2.T.astype(jnp.bfloat16)
    b2t = b2.reshape(H, 1)
    wh = jnp.concatenate([w_mu, w_sig], axis=1).T.astype(jnp.bfloat16)
    bh = jnp.concatenate([b_mu, b_sig], axis=1).reshape(2, 1)

    TB = min(_TB, B)
    TBS = TB // _NSPLIT
    grid = (pl.cdiv(B, TB),)

    def _x_spec(j):
        return pl.BlockSpec((TBS, S), lambda i, j=j: (_NSPLIT * i + j, 0))

    out = pl.pallas_call(
        _fused_policy_kernel,
        out_shape=jax.ShapeDtypeStruct((2, B), jnp.float32),
        grid=grid,
        in_specs=[_x_spec(j) for j in range(_NSPLIT)] + [
            pl.BlockSpec((S, H), lambda i: (0, 0)),
            pl.BlockSpec((H, 1), lambda i: (0, 0)),
            pl.BlockSpec((H, H), lambda i: (0, 0)),
            pl.BlockSpec((H, 1), lambda i: (0, 0)),
            pl.BlockSpec((2, H), lambda i: (0, 0)),
            pl.BlockSpec((2, 1), lambda i: (0, 0)),
        ],
        out_specs=pl.BlockSpec((2, TB), lambda i: (0, i)),
        compiler_params=pltpu.CompilerParams(
            dimension_semantics=("parallel",),
        ),
    )(*([x] * _NSPLIT), w1b, b1t, w2tb, b2t, wh, bh)

    mu = out[0, :].reshape(B, 1)
    sigma = out[1, :].reshape(B, 1)
    return mu, sigma


# speedup vs baseline: 1.0710x; 1.0710x over previous
import jax
import jax.numpy as jnp
from jax.experimental import pallas as pl
from jax.experimental.pallas import tpu as pltpu


def _fused_policy_kernel(x_ref, w1_ref, b1_ref, w2t_ref, b2_ref,
                         wh_ref, bh_ref, mu_ref, sig_ref):
    xb = x_ref[...].astype(jnp.bfloat16)

    h = jax.lax.dot_general(
        w1_ref[...], xb, (((0,), (1,)), ((), ())),
        preferred_element_type=jnp.float32) + b1_ref[...]
    h = jnp.maximum(h, 0.0).astype(jnp.bfloat16)

    h = jnp.dot(w2t_ref[...], h,
                preferred_element_type=jnp.float32) + b2_ref[...]
    h = jnp.maximum(h, 0.0).astype(jnp.bfloat16)

    z = jnp.dot(wh_ref[...], h,
                preferred_element_type=jnp.float32) + bh_ref[...]

    zm = z[0:1, :]
    zs = z[1:2, :]
    mu_ref[...] = jnp.tanh(zm) * 2.0
    sig_ref[...] = (jnp.maximum(zs, 0.0)
                    + jnp.log1p(jnp.exp(-jnp.abs(zs)))
                    + 1e-5)


def kernel(x, w1, b1, w2, b2, w_mu, b_mu, w_sig, b_sig):
    B, S = x.shape
    H = w1.shape[1]

    w1b = w1.astype(jnp.bfloat16)
    b1t = b1.reshape(H, 1)
    w2tb = w2.T.astype(jnp.bfloat16)
    b2t = b2.reshape(H, 1)
    wh = jnp.concatenate([w_mu, w_sig], axis=1).T.astype(jnp.bfloat16)
    bh = jnp.concatenate([b_mu, b_sig], axis=1).reshape(2, 1)

    TB = min(16384, B)
    grid = (pl.cdiv(B, TB),)

    mu2d, sig2d = pl.pallas_call(
        _fused_policy_kernel,
        out_shape=(jax.ShapeDtypeStruct((1, B), jnp.float32),
                   jax.ShapeDtypeStruct((1, B), jnp.float32)),
        grid=grid,
        in_specs=[
            pl.BlockSpec((TB, S), lambda i: (i, 0)),
            pl.BlockSpec((S, H), lambda i: (0, 0)),
            pl.BlockSpec((H, 1), lambda i: (0, 0)),
            pl.BlockSpec((H, H), lambda i: (0, 0)),
            pl.BlockSpec((H, 1), lambda i: (0, 0)),
            pl.BlockSpec((2, H), lambda i: (0, 0)),
            pl.BlockSpec((2, 1), lambda i: (0, 0)),
        ],
        out_specs=(pl.BlockSpec((1, TB), lambda i: (0, i)),
                   pl.BlockSpec((1, TB), lambda i: (0, i))),
        compiler_params=pltpu.CompilerParams(
            dimension_semantics=("parallel",),
        ),
    )(x, w1b, b1t, w2tb, b2t, wh, bh)

    mu = mu2d.reshape(B, 1)
    sigma = sig2d.reshape(B, 1)
    return mu, sigma
